# baseline (speedup 1.0000x reference)
import math

import numpy as np
import jax
import jax.numpy as jnp
from jax import lax
from jax.experimental import pallas as pl
from jax.experimental.pallas import tpu as pltpu
from jax.experimental.pallas import tpu_sc as plsc

_BATCH = 16384
_OUTPUT_DIM = 1000
_ROT_A = (13, 15, 26, 6)
_ROT_B = (17, 29, 16, 24)
_KS = (0, 42, (0x1BD11BDA ^ 42) & 0xFFFFFFFF)
_LOGP = float(np.float32(math.log(1.0 / _OUTPUT_DIM)))

_TC_ROWS = 12288
_SC_ROWS = _BATCH - _TC_ROWS

_NW = 32
_SC_ROWS_PER_W = _SC_ROWS // _NW
_SC_GROUPS = _SC_ROWS_PER_W // 16
_SC_UNROLL = 5

_TC_ROWS_PER_PROG = 6144
_TC_GRID = _TC_ROWS // _TC_ROWS_PER_PROG
_TC_UNROLL = 25


def _rotl(x, r):
    return (x << jnp.uint32(r)) | (x >> jnp.uint32(32 - r))


def _threefry_bits(x1_init_u32):
    x1 = x1_init_u32
    x0 = x1
    first = True
    for i in range(5):
        rots = _ROT_A if i % 2 == 0 else _ROT_B
        for r in rots:
            if first:
                first = False
            else:
                x0 = x0 + x1
            x1 = _rotl(x1, r)
            x1 = x1 ^ x0
        x0 = x0 + jnp.uint32(_KS[(i + 1) % 3])
        x1 = x1 + jnp.uint32((_KS[(i + 2) % 3] + i + 1) & 0xFFFFFFFF)
    return x0 ^ x1


_MMASK = 0xFFFFFE00
_HALF = _OUTPUT_DIM // 2


def _tree_max(vals):
    while len(vals) > 1:
        vals = [
            jnp.maximum(vals[2 * k], vals[2 * k + 1])
            for k in range(len(vals) // 2)
        ] + (vals[-1:] if len(vals) % 2 else [])
    return vals[0]


def _argmax_scan(base_u32, shape, unroll):
    def half_body(j_lo, p_off):
        def body(it, best):
            keys = []
            for jj in range(unroll):
                j = j_lo + it * unroll + jj
                bits = _threefry_bits(base_u32 + (j + 42).astype(jnp.uint32))
                bits_i = lax.bitcast_convert_type(bits, jnp.int32)
                keys.append(
                    (bits_i & jnp.int32(-512))
                    ^ ((p_off - j) + jnp.int32(-(2**31)))
                )
            return jnp.maximum(best, _tree_max(keys))

        return lax.fori_loop(
            0, _HALF // unroll, body, jnp.full(shape, -(2**31), jnp.int32)
        )

    best_a = half_body(0, 511)
    best_b = half_body(_HALF, 1011)
    m_a = best_a & jnp.int32(-512)
    m_b = best_b & jnp.int32(-512)
    use_b = m_b > m_a
    j_a = 511 - (best_a & jnp.int32(511))
    j_b = 1011 - (best_b & jnp.int32(511))
    return jnp.where(use_b, j_b, j_a)


def _tc_kernel(actions_ref):
    shape = (_TC_ROWS_PER_PROG // 1024, 8, 128)
    row0 = (
        lax.broadcasted_iota(jnp.int32, shape, 0) * 1024
        + lax.broadcasted_iota(jnp.int32, shape, 1) * 128
        + lax.broadcasted_iota(jnp.int32, shape, 2)
    )

    def chunk(p, _):
        row = row0 + p * _TC_ROWS_PER_PROG
        base = (row * _OUTPUT_DIM).astype(jnp.uint32)
        actions_ref[pl.ds(p, 1)] = _argmax_scan(base, shape, _TC_UNROLL)[None]
        return 0

    lax.fori_loop(0, _TC_GRID, chunk, 0)


def _tc_sample():
    actions = pl.pallas_call(
        _tc_kernel,
        out_shape=jax.ShapeDtypeStruct(
            (_TC_GRID, _TC_ROWS_PER_PROG // 1024, 8, 128), jnp.int32
        ),
    )()
    return actions.reshape(-1)


_LP_PER_W = _BATCH // _NW


def _sc_body(actions_hbm, lp_hbm, actions_v, lp_v):
    cid = lax.axis_index("c")
    sid = lax.axis_index("s")
    wid = sid * 2 + cid
    row0 = _TC_ROWS + wid * _SC_ROWS_PER_W
    lane = lax.broadcasted_iota(jnp.int32, (16,), 0)

    def lp_body(g, _):
        lp_v[pl.ds(g * 16, 16)] = jnp.full((16,), _LOGP, jnp.float32)
        return 0

    lax.fori_loop(0, _LP_PER_W // 16, lp_body, 0)

    def g_body(g, _):
        rows = row0 + g * 16 + lane
        base = (rows * _OUTPUT_DIM).astype(jnp.uint32)
        actions_v[pl.ds(g * 16, 16)] = _argmax_scan(base, (16,), _SC_UNROLL)
        return 0

    lax.fori_loop(0, _SC_GROUPS, g_body, 0)
    out0 = wid * _SC_ROWS_PER_W
    pltpu.sync_copy(actions_v, actions_hbm.at[pl.ds(out0, _SC_ROWS_PER_W)])
    pltpu.sync_copy(lp_v, lp_hbm.at[pl.ds(wid * _LP_PER_W, _LP_PER_W)])


def _sc_sample():
    mesh = plsc.VectorSubcoreMesh(
        core_axis_name="c", subcore_axis_name="s", num_cores=2, num_subcores=16
    )
    f = pl.kernel(
        _sc_body,
        out_type=(
            jax.ShapeDtypeStruct((_SC_ROWS,), jnp.int32),
            jax.ShapeDtypeStruct((_BATCH,), jnp.float32),
        ),
        mesh=mesh,
        scratch_types=[
            pltpu.VMEM((_SC_ROWS_PER_W,), jnp.int32),
            pltpu.VMEM((_LP_PER_W,), jnp.float32),
        ],
    )
    return f()


@jax.jit
def _sample():
    sc_actions, log_probs = _sc_sample()
    tc_actions = _tc_sample()
    actions = jnp.concatenate([tc_actions, sc_actions])
    return actions, log_probs


def kernel(state):
    batch_size = state.shape[0]
    actions, log_probs = _sample()
    return actions[:batch_size], log_probs[:batch_size]


# speedup vs baseline: 1.0058x; 1.0058x over previous
import math

import numpy as np
import jax
import jax.numpy as jnp
from jax import lax
from jax.experimental import pallas as pl
from jax.experimental.pallas import tpu as pltpu
from jax.experimental.pallas import tpu_sc as plsc

_BATCH = 16384
_OUTPUT_DIM = 1000
_ROT_A = (13, 15, 26, 6)
_ROT_B = (17, 29, 16, 24)
_KS = (0, 42, (0x1BD11BDA ^ 42) & 0xFFFFFFFF)
_LOGP = float(np.float32(math.log(1.0 / _OUTPUT_DIM)))

_TC_ROWS = 12288
_SC_ROWS = _BATCH - _TC_ROWS

_NW = 32
_SC_ROWS_PER_W = _SC_ROWS // _NW
_SC_GROUPS = _SC_ROWS_PER_W // 16
_SC_UNROLL = 4

_TC_ROWS_PER_PROG = 4096
_TC_GRID = _TC_ROWS // _TC_ROWS_PER_PROG
_TC_UNROLL = 25


def _rotl(x, r):
    return (x << jnp.uint32(r)) | (x >> jnp.uint32(32 - r))


def _threefry_bits(x1_init_u32):
    x1 = x1_init_u32
    x0 = x1
    first = True
    for i in range(5):
        rots = _ROT_A if i % 2 == 0 else _ROT_B
        for r in rots:
            if first:
                first = False
            else:
                x0 = x0 + x1
            x1 = _rotl(x1, r)
            x1 = x1 ^ x0
        x0 = x0 + jnp.uint32(_KS[(i + 1) % 3])
        x1 = x1 + jnp.uint32((_KS[(i + 2) % 3] + i + 1) & 0xFFFFFFFF)
    return x0 ^ x1


_MMASK = 0xFFFFFE00
_HALF = _OUTPUT_DIM // 2


def _tree_max(vals):
    while len(vals) > 1:
        vals = [
            jnp.maximum(vals[2 * k], vals[2 * k + 1])
            for k in range(len(vals) // 2)
        ] + (vals[-1:] if len(vals) % 2 else [])
    return vals[0]


def _argmax_scan(base_u32, shape, unroll):
    def half_body(j_lo, p_off):
        def body(it, best):
            keys = []
            for jj in range(unroll):
                j = j_lo + it * unroll + jj
                bits = _threefry_bits(base_u32 + (j + 42).astype(jnp.uint32))
                bits_i = lax.bitcast_convert_type(bits, jnp.int32)
                keys.append(
                    (bits_i & jnp.int32(-512))
                    ^ ((p_off - j) + jnp.int32(-(2**31)))
                )
            return jnp.maximum(best, _tree_max(keys))

        return lax.fori_loop(
            0, _HALF // unroll, body, jnp.full(shape, -(2**31), jnp.int32)
        )

    best_a = half_body(0, 511)
    best_b = half_body(_HALF, 1011)
    m_a = best_a & jnp.int32(-512)
    m_b = best_b & jnp.int32(-512)
    use_b = m_b > m_a
    j_a = 511 - (best_a & jnp.int32(511))
    j_b = 1011 - (best_b & jnp.int32(511))
    return jnp.where(use_b, j_b, j_a)


def _tc_kernel(actions_ref):
    shape = (_TC_ROWS_PER_PROG // 1024, 8, 128)
    row0 = (
        lax.broadcasted_iota(jnp.int32, shape, 0) * 1024
        + lax.broadcasted_iota(jnp.int32, shape, 1) * 128
        + lax.broadcasted_iota(jnp.int32, shape, 2)
    )

    def chunk(p, _):
        row = row0 + p * _TC_ROWS_PER_PROG
        base = (row * _OUTPUT_DIM).astype(jnp.uint32)
        actions_ref[pl.ds(p, 1)] = _argmax_scan(base, shape, _TC_UNROLL)[None]
        return 0

    lax.fori_loop(0, _TC_GRID, chunk, 0)


def _tc_sample():
    actions = pl.pallas_call(
        _tc_kernel,
        out_shape=jax.ShapeDtypeStruct(
            (_TC_GRID, _TC_ROWS_PER_PROG // 1024, 8, 128), jnp.int32
        ),
    )()
    return actions.reshape(-1)


_LP_PER_W = _BATCH // _NW


def _sc_body(actions_hbm, lp_hbm, actions_v, lp_v):
    cid = lax.axis_index("c")
    sid = lax.axis_index("s")
    wid = sid * 2 + cid
    row0 = _TC_ROWS + wid * _SC_ROWS_PER_W
    lane = lax.broadcasted_iota(jnp.int32, (16,), 0)

    def lp_body(g, _):
        lp_v[pl.ds(g * 16, 16)] = jnp.full((16,), _LOGP, jnp.float32)
        return 0

    lax.fori_loop(0, _LP_PER_W // 16, lp_body, 0)

    def g_body(g, _):
        rows = row0 + g * 16 + lane
        base = (rows * _OUTPUT_DIM).astype(jnp.uint32)
        actions_v[pl.ds(g * 16, 16)] = _argmax_scan(base, (16,), _SC_UNROLL)
        return 0

    lax.fori_loop(0, _SC_GROUPS, g_body, 0)
    out0 = wid * _SC_ROWS_PER_W
    pltpu.sync_copy(actions_v, actions_hbm.at[pl.ds(out0, _SC_ROWS_PER_W)])
    pltpu.sync_copy(lp_v, lp_hbm.at[pl.ds(wid * _LP_PER_W, _LP_PER_W)])


def _sc_sample():
    mesh = plsc.VectorSubcoreMesh(
        core_axis_name="c", subcore_axis_name="s", num_cores=2, num_subcores=16
    )
    f = pl.kernel(
        _sc_body,
        out_type=(
            jax.ShapeDtypeStruct((_SC_ROWS,), jnp.int32),
            jax.ShapeDtypeStruct((_BATCH,), jnp.float32),
        ),
        mesh=mesh,
        scratch_types=[
            pltpu.VMEM((_SC_ROWS_PER_W,), jnp.int32),
            pltpu.VMEM((_LP_PER_W,), jnp.float32),
        ],
    )
    return f()


@jax.jit
def _sample():
    sc_actions, log_probs = _sc_sample()
    tc_actions = _tc_sample()
    actions = jnp.concatenate([tc_actions, sc_actions])
    return actions, log_probs


def kernel(state):
    batch_size = state.shape[0]
    actions, log_probs = _sample()
    return actions[:batch_size], log_probs[:batch_size]
